# baseline (speedup 1.0000x reference)
import jax
import jax.numpy as jnp
from jax.experimental import pallas as pl
from jax.experimental.pallas import tpu as pltpu

BATCH, SEQ, MODEL_DIM = 4, 4096, 1024
ROWS = BATCH * SEQ
ROW_BLK = 1024


def _body(m_ref, s_ref, e_ref, o_ref):
    m = m_ref[...] != 0
    o_ref[...] = jnp.where(m, e_ref[...], s_ref[...])


def kernel(seqs, temporal_mask, temporal_mask_embed):
    mask_col = temporal_mask.reshape(ROWS, 1).astype(jnp.int32)
    seqs2 = seqs.reshape(ROWS, MODEL_DIM)
    embed2d = temporal_mask_embed.reshape(1, MODEL_DIM)

    out = pl.pallas_call(
        _body,
        grid=(ROWS // ROW_BLK,),
        in_specs=[
            pl.BlockSpec((ROW_BLK, 1), lambda i: (i, 0)),
            pl.BlockSpec((ROW_BLK, MODEL_DIM), lambda i: (i, 0)),
            pl.BlockSpec((1, MODEL_DIM), lambda i: (0, 0)),
        ],
        out_specs=pl.BlockSpec((ROW_BLK, MODEL_DIM), lambda i: (i, 0)),
        out_shape=jax.ShapeDtypeStruct((ROWS, MODEL_DIM), seqs.dtype),
    )(mask_col, seqs2, embed2d)
    return (out.reshape(BATCH, SEQ, MODEL_DIM), temporal_mask)


# speedup vs baseline: 1.1147x; 1.1147x over previous
import jax
import jax.numpy as jnp
from jax.experimental import pallas as pl
from jax.experimental.pallas import tpu as pltpu

BATCH, SEQ, MODEL_DIM = 4, 4096, 1024
ROWS = BATCH * SEQ
G0 = ROWS // 128
MROWS = 8


def _body(mt_ref, s_ref, e_ref, o_ref):
    msub = mt_ref[0]
    e = e_ref[...]
    for j in range(MROWS):
        mj = msub[:, j:j + 1] != 0
        o_ref[j] = jnp.where(mj, e, s_ref[j])


def kernel(seqs, temporal_mask, temporal_mask_embed):
    mask_t3 = (temporal_mask.reshape(G0 // MROWS, MROWS, 128)
               .astype(jnp.int32).transpose(0, 2, 1))
    seqs3 = seqs.reshape(G0, 128, MODEL_DIM)
    embed2d = temporal_mask_embed.reshape(1, MODEL_DIM)

    out = pl.pallas_call(
        _body,
        grid=(G0 // MROWS,),
        in_specs=[
            pl.BlockSpec((1, 128, MROWS), lambda i: (i, 0, 0)),
            pl.BlockSpec((MROWS, 128, MODEL_DIM), lambda i: (i, 0, 0)),
            pl.BlockSpec((1, MODEL_DIM), lambda i: (0, 0)),
        ],
        out_specs=pl.BlockSpec((MROWS, 128, MODEL_DIM), lambda i: (i, 0, 0)),
        out_shape=jax.ShapeDtypeStruct((G0, 128, MODEL_DIM), seqs.dtype),
    )(mask_t3, seqs3, embed2d)
    return (out.reshape(BATCH, SEQ, MODEL_DIM), temporal_mask)
